# Initial kernel scaffold; baseline (speedup 1.0000x reference)
import jax
import jax.numpy as jnp
from jax.experimental import pallas as pl


def kernel(x_src, y_src, fa_in_w, fa_in_b, fa_out_w, fa_out_b, idx_qw, idx_kw, idx_ow, mla_qw, mla_down, mla_up, mla_out, n1_g, n1_b, n2_g, n2_b, n3_g, n3_b, mlp1_w, mlp1_b, mlp2_w, mlp2_b, fe_w, fe_b, te_w, te_b, dec1_w, dec1_b, dec2_w, dec2_b):
    raise NotImplementedError("write your pallas kernel here")


# speedup vs baseline: 7.2621x; 7.2621x over previous
import jax
import jax.numpy as jnp
from jax.experimental import pallas as pl

E = 192
NHEAD = 4
HD = 48
MLPD = 384
TOPK = 16
SPLIT = 256
ROWS = 512
COLS = 16
CT = 17
IDXH = 4
IDXD = 8
LAT = 96
RB = 15
CHUNK = RB * CT
NCH = 35
NTOK = ROWS * CT
PADTOK = NCH * CHUNK

_HI = jax.lax.Precision.HIGHEST


def _dt(a, b, prec=None):
    return jax.lax.dot_general(a, b, (((1,), (1,)), ((), ())),
                               precision=prec, preferred_element_type=jnp.float32)


def _dot(a, b, prec=None):
    return jax.lax.dot_general(a, b, (((1,), (0,)), ((), ())),
                               precision=prec, preferred_element_type=jnp.float32)


def _ln(x, g, b):
    m = jnp.mean(x, axis=-1, keepdims=True)
    v = jnp.mean((x - m) ** 2, axis=-1, keepdims=True)
    return (x - m) / jnp.sqrt(v + 1e-5) * g + b


def _gelu(x):
    return 0.5 * x * (1.0 + jax.lax.erf(x / jnp.sqrt(jnp.float32(2.0))))


def _embed_body(x_ref, y_ref, few_ref, feb_ref, tew_ref, teb_ref, xo_ref, yo_ref):
    xo_ref[...] = x_ref[...] * few_ref[...] + feb_ref[...]
    yo_ref[...] = y_ref[...] * tew_ref[...] + teb_ref[...]


def _embed_call(x2d, y2d, few, feb, tew, teb):
    return pl.pallas_call(
        _embed_body,
        out_shape=(jax.ShapeDtypeStruct((ROWS * COLS, E), jnp.float32),
                   jax.ShapeDtypeStruct((ROWS, E), jnp.float32)),
    )(x2d, y2d, few, feb, tew, teb)


def _fa_body(tok_ref, wq_ref, wk_ref, wv_ref, bq_ref, bk_ref, bv_ref,
             wo_ref, bo_ref, g_ref, b_ref, rc_ref, rr_ref, out_ref):
    x = tok_ref[0]
    q = _dt(x, wq_ref[...]) + bq_ref[...]
    k = _dt(x, wk_ref[...]) + bk_ref[...]
    v = _dt(x, wv_ref[...]) + bv_ref[...]
    blk = rc_ref[...] == rr_ref[...]
    li = jax.lax.broadcasted_iota(jnp.int32, (1, E), 1)
    acc = jnp.zeros((CHUNK, E), jnp.float32)
    for h in range(NHEAD):
        hm = ((li >= h * HD) & (li < (h + 1) * HD)).astype(jnp.float32)
        s = _dt(q * hm, k) / jnp.sqrt(jnp.float32(HD))
        s = jnp.where(blk, s, jnp.float32(-1e30))
        s = s - jnp.max(s, axis=-1, keepdims=True)
        p = jnp.exp(s)
        p = p / jnp.sum(p, axis=-1, keepdims=True)
        acc = acc + _dot(p, v * hm)
    o = _dt(acc, wo_ref[...]) + bo_ref[...] + x
    out_ref[0] = _ln(o, g_ref[...], b_ref[...])


def _fa_call(tok3, wq, wk, wv, bq, bk, bv, wo, bo, g, b, rc, rr):
    def full(a):
        shp = a.shape
        return pl.BlockSpec(shp, lambda i: tuple(0 for _ in shp))
    consts = (wq, wk, wv, bq, bk, bv, wo, bo, g, b, rc, rr)
    return pl.pallas_call(
        _fa_body,
        grid=(NCH,),
        in_specs=[pl.BlockSpec((1, CHUNK, E), lambda i: (i, 0, 0))] + [full(a) for a in consts],
        out_specs=pl.BlockSpec((1, CHUNK, E), lambda i: (i, 0, 0)),
        out_shape=jax.ShapeDtypeStruct((NCH, CHUNK, E), jnp.float32),
    )(tok3, *consts)


def _bcd_body(st_ref, qw_ref, kw_ref, ow_ref, mqw_ref, down_ref, upk_ref,
              upv_ref, mout_ref, n2g_ref, n2b_ref, n3g_ref, n3b_ref,
              m1w_ref, m1b_ref, m2w_ref, m2b_ref, out_ref):
    x = st_ref[0]
    keys = x[:SPLIT]

    qi = _dt(x, qw_ref[...])
    ki = _dt(keys, kw_ref[...])
    li = jax.lax.broadcasted_iota(jnp.int32, (1, IDXH * IDXD), 1)
    hms = [((li >= h * IDXD) & (li < (h + 1) * IDXD)).astype(jnp.float32)
           for h in range(IDXH)]
    k_abs = jnp.abs(ki)
    ks = [(jnp.max(k_abs * hms[h], keepdims=True) + 1e-6) / 127.0
          for h in range(IDXH)]
    k_scale_vec = sum(ks[h] * hms[h] for h in range(IDXH))
    k_q = jnp.clip(jnp.round(ki / k_scale_vec), -127.0, 127.0)

    halves = []
    for off in (0, SPLIT):
        qh = qi[off:off + SPLIT]
        q_abs = jnp.abs(qh)
        qs = [(jnp.max(q_abs * hms[h], keepdims=True) + 1e-6) / 127.0
              for h in range(IDXH)]
        q_scale_vec = sum(qs[h] * hms[h] for h in range(IDXH))
        q_q = jnp.clip(jnp.round(qh / q_scale_vec), -127.0, 127.0)
        red = jnp.zeros((SPLIT, SPLIT), jnp.float32)
        for h in range(IDXH):
            s = _dt(q_q * hms[h], k_q)
            red = red + jax.nn.relu(s * (qs[h] * ks[h])) * ow_ref[0:1, h:h + 1]
        halves.append(red)
    red = jnp.concatenate(halves, axis=0)

    iot = jax.lax.broadcasted_iota(jnp.int32, (ROWS, SPLIT), 1)
    cur = red
    mask = jnp.zeros((ROWS, SPLIT), jnp.float32)
    for _ in range(TOPK):
        m = jnp.max(cur, axis=-1, keepdims=True)
        cand = jnp.where(cur == m, iot, jnp.int32(1 << 30))
        idx = jnp.min(cand, axis=-1, keepdims=True)
        oh = iot == idx
        mask = jnp.where(oh, 1.0, mask)
        cur = jnp.where(oh, jnp.float32(-1e30), cur)

    c = _dt(keys, down_ref[...])
    k_all = _dt(c, upk_ref[...])
    v_all = _dt(c, upv_ref[...])
    qm = _dt(x, mqw_ref[...])
    liE = jax.lax.broadcasted_iota(jnp.int32, (1, E), 1)
    o = jnp.zeros((ROWS, E), jnp.float32)
    sel = mask > 0.5
    for h in range(NHEAD):
        hm = ((liE >= h * HD) & (liE < (h + 1) * HD)).astype(jnp.float32)
        s = _dt(qm * hm, k_all * hm, _HI) / jnp.sqrt(jnp.float32(HD))
        s = jnp.where(sel, s, jnp.float32(-1e30))
        s = s - jnp.max(s, axis=-1, keepdims=True)
        p = jnp.exp(s)
        p = p / jnp.sum(p, axis=-1, keepdims=True)
        o = o + _dot(p, v_all * hm, _HI)
    attn = _dt(o, mout_ref[...])
    src2 = _ln(x + attn, n2g_ref[...], n2b_ref[...])

    h1 = _gelu(_dt(src2, m1w_ref[...]) + m1b_ref[...])
    src3 = _dt(h1, m2w_ref[...]) + m2b_ref[...] + src2
    out_ref[0] = _ln(src3, n3g_ref[...], n3b_ref[...])


def _bcd_call(st, qw, kw, ow, mqw, down, upk, upv, mout,
              n2g, n2b, n3g, n3b, m1w, m1b, m2w, m2b):
    def full(a):
        shp = a.shape
        return pl.BlockSpec(shp, lambda i: tuple(0 for _ in shp))
    consts = (qw, kw, ow, mqw, down, upk, upv, mout,
              n2g, n2b, n3g, n3b, m1w, m1b, m2w, m2b)
    return pl.pallas_call(
        _bcd_body,
        grid=(CT,),
        in_specs=[pl.BlockSpec((1, ROWS, E), lambda f: (f, 0, 0))] + [full(a) for a in consts],
        out_specs=pl.BlockSpec((1, ROWS, E), lambda f: (f, 0, 0)),
        out_shape=jax.ShapeDtypeStruct((CT, ROWS, E), jnp.float32),
    )(st, *consts)


def _dec_body(x_ref, w1_ref, b1_ref, w2_ref, b2_ref, out_ref):
    h = _gelu(_dt(x_ref[...], w1_ref[...]) + b1_ref[...])
    out_ref[...] = _dt(h, w2_ref[...]) + b2_ref[...]


def _dec_call(x, w1, b1, w2, b2):
    return pl.pallas_call(
        _dec_body,
        out_shape=jax.ShapeDtypeStruct((ROWS - SPLIT, 2), jnp.float32),
    )(x, w1, b1, w2, b2)


def kernel(x_src, y_src, fa_in_w, fa_in_b, fa_out_w, fa_out_b,
           idx_qw, idx_kw, idx_ow, mla_qw, mla_down, mla_up, mla_out,
           n1_g, n1_b, n2_g, n2_b, n3_g, n3_b,
           mlp1_w, mlp1_b, mlp2_w, mlp2_b,
           fe_w, fe_b, te_w, te_b, dec1_w, dec1_b, dec2_w, dec2_b):
    f32 = jnp.float32
    x2d = x_src.reshape(ROWS * COLS, 1)
    y2d = jnp.concatenate([y_src[0], jnp.zeros((ROWS - SPLIT, 1), f32)], axis=0)
    xe, ye = _embed_call(x2d, y2d, fe_w.reshape(1, E), fe_b.reshape(1, E),
                         te_w.reshape(1, E), te_b.reshape(1, E))
    src = jnp.concatenate([xe.reshape(ROWS, COLS, E), ye.reshape(ROWS, 1, E)],
                          axis=1)
    rc = (jnp.arange(CHUNK, dtype=jnp.int32) // CT).reshape(CHUNK, 1)
    rr = rc.reshape(1, CHUNK)
    st = None
    for l in range(2):
        tok = src.reshape(NTOK, E)
        tok3 = jnp.pad(tok, ((0, PADTOK - NTOK), (0, 0))).reshape(NCH, CHUNK, E)
        ln1 = _fa_call(tok3,
                       fa_in_w[l, 0:E], fa_in_w[l, E:2 * E], fa_in_w[l, 2 * E:3 * E],
                       fa_in_b[l, 0:E].reshape(1, E), fa_in_b[l, E:2 * E].reshape(1, E),
                       fa_in_b[l, 2 * E:3 * E].reshape(1, E),
                       fa_out_w[l], fa_out_b[l].reshape(1, E),
                       n1_g[l].reshape(1, E), n1_b[l].reshape(1, E), rc, rr)
        st_in = ln1.reshape(PADTOK, E)[:NTOK].reshape(ROWS, CT, E).transpose(1, 0, 2)
        st = _bcd_call(st_in, idx_qw[l], idx_kw[l], idx_ow[l].reshape(1, IDXH),
                       mla_qw[l], mla_down[l], mla_up[l][:E], mla_up[l][E:],
                       mla_out[l],
                       n2_g[l].reshape(1, E), n2_b[l].reshape(1, E),
                       n3_g[l].reshape(1, E), n3_b[l].reshape(1, E),
                       mlp1_w[l], mlp1_b[l].reshape(1, MLPD),
                       mlp2_w[l], mlp2_b[l].reshape(1, E))
        if l == 0:
            src = st.transpose(1, 0, 2)
    tgt = st[CT - 1, SPLIT:]
    out = _dec_call(tgt, dec1_w, dec1_b.reshape(1, MLPD),
                    dec2_w, dec2_b.reshape(1, 2))
    return out.reshape(1, ROWS - SPLIT, 2)


# speedup vs baseline: 9.9668x; 1.3724x over previous
import jax
import jax.numpy as jnp
from jax.experimental import pallas as pl

E = 192
NHEAD = 4
HD = 48
MLPD = 384
TOPK = 16
SPLIT = 256
ROWS = 512
COLS = 16
CT = 17
IDXH = 4
IDXD = 8
LAT = 96
RB = 16
CHUNK = RB * CT
NCH = ROWS // RB
PADR = ROWS


def _dt(a, b, prec=None):
    return jax.lax.dot_general(a, b, (((1,), (1,)), ((), ())),
                               precision=prec, preferred_element_type=jnp.float32)


def _dot(a, b, prec=None):
    return jax.lax.dot_general(a, b, (((1,), (0,)), ((), ())),
                               precision=prec, preferred_element_type=jnp.float32)


def _split(a):
    hi = a.astype(jnp.bfloat16).astype(jnp.float32)
    return hi, a - hi


def _ln(x, g, b):
    m = jnp.mean(x, axis=-1, keepdims=True)
    v = jnp.mean((x - m) ** 2, axis=-1, keepdims=True)
    return (x - m) / jnp.sqrt(v + 1e-5) * g + b


def _gelu(x):
    return 0.5 * x * (1.0 + jax.lax.erf(x / jnp.sqrt(jnp.float32(2.0))))


def _embed_body(x_ref, w_ref, b_ref, out_ref):
    out_ref[0] = x_ref[0] * w_ref[0] + b_ref[0]


def _embed_call(xcols, w, b):
    return pl.pallas_call(
        _embed_body,
        grid=(CT,),
        in_specs=[pl.BlockSpec((1, PADR, 1), lambda f: (f, 0, 0)),
                  pl.BlockSpec((1, 1, E), lambda f: (f, 0, 0)),
                  pl.BlockSpec((1, 1, E), lambda f: (f, 0, 0))],
        out_specs=pl.BlockSpec((1, PADR, E), lambda f: (f, 0, 0)),
        out_shape=jax.ShapeDtypeStruct((CT, PADR, E), jnp.float32),
    )(xcols, w, b)


def _fa_body(tok_ref, wq_ref, wk_ref, wv_ref, bq_ref, bk_ref, bv_ref,
             wo_ref, bo_ref, g_ref, b_ref, rc_ref, rr_ref, out_ref):
    x = tok_ref[...].reshape(CHUNK, E)
    q = _dt(x, wq_ref[...]) + bq_ref[...]
    k = _dt(x, wk_ref[...]) + bk_ref[...]
    v = _dt(x, wv_ref[...]) + bv_ref[...]
    blk = rc_ref[...] == rr_ref[...]
    li = jax.lax.broadcasted_iota(jnp.int32, (1, E), 1)
    acc = jnp.zeros((CHUNK, E), jnp.float32)
    for h in range(NHEAD):
        hm = ((li >= h * HD) & (li < (h + 1) * HD)).astype(jnp.float32)
        s = _dt(q * hm, k) / jnp.sqrt(jnp.float32(HD))
        p = jnp.where(blk, jnp.exp(s), 0.0)
        p = p / jnp.sum(p, axis=-1, keepdims=True)
        acc = acc + _dot(p, v * hm)
    o = _dt(acc, wo_ref[...]) + bo_ref[...] + x
    out_ref[...] = _ln(o, g_ref[...], b_ref[...]).reshape(CT, RB, E)


def _fa_call(st, wq, wk, wv, bq, bk, bv, wo, bo, g, b, rc, rr):
    def full(a):
        shp = a.shape
        return pl.BlockSpec(shp, lambda i: tuple(0 for _ in shp))
    consts = (wq, wk, wv, bq, bk, bv, wo, bo, g, b, rc, rr)
    return pl.pallas_call(
        _fa_body,
        grid=(NCH,),
        in_specs=[pl.BlockSpec((CT, RB, E), lambda i: (0, i, 0))] + [full(a) for a in consts],
        out_specs=pl.BlockSpec((CT, RB, E), lambda i: (0, i, 0)),
        out_shape=jax.ShapeDtypeStruct((CT, PADR, E), jnp.float32),
    )(st, *consts)


def _bcd_body(st_ref, qw_ref, kw_ref, ow_ref, mqw_ref, down_ref, upk_ref,
              upv_ref, mout_ref, n2g_ref, n2b_ref, n3g_ref, n3b_ref,
              m1w_ref, m1b_ref, m2w_ref, m2b_ref, out_ref):
    x = st_ref[0][:ROWS] if PADR > ROWS else st_ref[0]
    keys = x[:SPLIT]

    qi = _dt(x, qw_ref[...])
    ki = _dt(keys, kw_ref[...])
    li = jax.lax.broadcasted_iota(jnp.int32, (1, IDXH * IDXD), 1)
    hms = [((li >= h * IDXD) & (li < (h + 1) * IDXD)).astype(jnp.float32)
           for h in range(IDXH)]
    k_abs = jnp.abs(ki)
    ks = [(jnp.max(k_abs * hms[h], keepdims=True) + 1e-6) / 127.0
          for h in range(IDXH)]
    k_scale_vec = sum(ks[h] * hms[h] for h in range(IDXH))
    k_q = jnp.clip(jnp.round(ki / k_scale_vec), -127.0, 127.0)

    halves = []
    for off in (0, SPLIT):
        qh = qi[off:off + SPLIT]
        q_abs = jnp.abs(qh)
        qs = [(jnp.max(q_abs * hms[h], keepdims=True) + 1e-6) / 127.0
              for h in range(IDXH)]
        q_scale_vec = sum(qs[h] * hms[h] for h in range(IDXH))
        q_q = jnp.clip(jnp.round(qh / q_scale_vec), -127.0, 127.0)
        red = jnp.zeros((SPLIT, SPLIT), jnp.float32)
        for h in range(IDXH):
            s = _dt(q_q * hms[h], k_q)
            red = red + (qs[h] * ks[h] * ow_ref[0:1, h:h + 1]) * jax.nn.relu(s)
        halves.append(red)
    red = jnp.concatenate(halves, axis=0)

    iot = jax.lax.broadcasted_iota(jnp.int32, (ROWS, SPLIT), 1)
    cur = red
    mask = jnp.zeros((ROWS, SPLIT), jnp.float32)
    for _ in range(TOPK):
        idx = jnp.argmax(cur, axis=-1).reshape(ROWS, 1)
        oh = iot == idx
        mask = jnp.where(oh, 1.0, mask)
        cur = jnp.where(oh, jnp.float32(-1e30), cur)

    c = _dt(keys, down_ref[...])
    k_all = _dt(c, upk_ref[...])
    v_all = _dt(c, upv_ref[...])
    qm = _dt(x, mqw_ref[...])
    liE = jax.lax.broadcasted_iota(jnp.int32, (1, E), 1)
    o = jnp.zeros((ROWS, E), jnp.float32)
    sel = mask > 0.5
    qm_hi, qm_lo = _split(qm)
    ka_hi, ka_lo = _split(k_all)
    va_hi, va_lo = _split(v_all)
    for h in range(NHEAD):
        hm = ((liE >= h * HD) & (liE < (h + 1) * HD)).astype(jnp.float32)
        a_hi = qm_hi * hm
        a_lo = qm_lo * hm
        s = (_dt(a_hi, ka_hi) + (_dt(a_lo, ka_hi) + _dt(a_hi, ka_lo)))
        s = s / jnp.sqrt(jnp.float32(HD))
        p = jnp.where(sel, jnp.exp(s), 0.0)
        p = p / jnp.sum(p, axis=-1, keepdims=True)
        p_hi, p_lo = _split(p)
        vm_hi = va_hi * hm
        vm_lo = va_lo * hm
        o = o + (_dot(p_hi, vm_hi) + (_dot(p_lo, vm_hi) + _dot(p_hi, vm_lo)))
    attn = _dt(o, mout_ref[...])
    src2 = _ln(x + attn, n2g_ref[...], n2b_ref[...])

    h1 = _gelu(_dt(src2, m1w_ref[...]) + m1b_ref[...])
    src3 = _dt(h1, m2w_ref[...]) + m2b_ref[...] + src2
    out_ref[0, :ROWS] = _ln(src3, n3g_ref[...], n3b_ref[...])
    if PADR > ROWS:
        out_ref[0, ROWS:] = jnp.zeros((PADR - ROWS, E), jnp.float32)


def _bcd_call(st, qw, kw, ow, mqw, down, upk, upv, mout,
              n2g, n2b, n3g, n3b, m1w, m1b, m2w, m2b):
    def full(a):
        shp = a.shape
        return pl.BlockSpec(shp, lambda i: tuple(0 for _ in shp))
    consts = (qw, kw, ow, mqw, down, upk, upv, mout,
              n2g, n2b, n3g, n3b, m1w, m1b, m2w, m2b)
    return pl.pallas_call(
        _bcd_body,
        grid=(CT,),
        in_specs=[pl.BlockSpec((1, PADR, E), lambda f: (f, 0, 0))] + [full(a) for a in consts],
        out_specs=pl.BlockSpec((1, PADR, E), lambda f: (f, 0, 0)),
        out_shape=jax.ShapeDtypeStruct((CT, PADR, E), jnp.float32),
    )(st, *consts)


def _dec_body(x_ref, w1_ref, b1_ref, w2_ref, b2_ref, out_ref):
    h = _gelu(_dt(x_ref[...], w1_ref[...]) + b1_ref[...])
    out_ref[...] = _dt(h, w2_ref[...]) + b2_ref[...]


def _dec_call(x, w1, b1, w2, b2):
    return pl.pallas_call(
        _dec_body,
        out_shape=jax.ShapeDtypeStruct((ROWS - SPLIT, 2), jnp.float32),
    )(x, w1, b1, w2, b2)


def kernel(x_src, y_src, fa_in_w, fa_in_b, fa_out_w, fa_out_b,
           idx_qw, idx_kw, idx_ow, mla_qw, mla_down, mla_up, mla_out,
           n1_g, n1_b, n2_g, n2_b, n3_g, n3_b,
           mlp1_w, mlp1_b, mlp2_w, mlp2_b,
           fe_w, fe_b, te_w, te_b, dec1_w, dec1_b, dec2_w, dec2_b):
    f32 = jnp.float32
    y_full = jnp.concatenate([y_src[0], jnp.zeros((ROWS - SPLIT, 1), f32)], axis=0)
    xcols = jnp.pad(jnp.concatenate([x_src[0], y_full], axis=1).T,
                    ((0, 0), (0, PADR - ROWS))).reshape(CT, PADR, 1)
    w_emb = jnp.concatenate([jnp.tile(fe_w.reshape(1, E), (COLS, 1)),
                             te_w.reshape(1, E)], axis=0).reshape(CT, 1, E)
    b_emb = jnp.concatenate([jnp.tile(fe_b.reshape(1, E), (COLS, 1)),
                             te_b.reshape(1, E)], axis=0).reshape(CT, 1, E)
    st = _embed_call(xcols, w_emb, b_emb)
    rc = (jnp.arange(CHUNK, dtype=jnp.int32) % RB).reshape(CHUNK, 1)
    rr = rc.reshape(1, CHUNK)
    for l in range(2):
        st = _fa_call(st,
                      fa_in_w[l, 0:E], fa_in_w[l, E:2 * E], fa_in_w[l, 2 * E:3 * E],
                      fa_in_b[l, 0:E].reshape(1, E), fa_in_b[l, E:2 * E].reshape(1, E),
                      fa_in_b[l, 2 * E:3 * E].reshape(1, E),
                      fa_out_w[l], fa_out_b[l].reshape(1, E),
                      n1_g[l].reshape(1, E), n1_b[l].reshape(1, E), rc, rr)
        st = _bcd_call(st, idx_qw[l], idx_kw[l], idx_ow[l].reshape(1, IDXH),
                       mla_qw[l], mla_down[l], mla_up[l][:E], mla_up[l][E:],
                       mla_out[l],
                       n2_g[l].reshape(1, E), n2_b[l].reshape(1, E),
                       n3_g[l].reshape(1, E), n3_b[l].reshape(1, E),
                       mlp1_w[l], mlp1_b[l].reshape(1, MLPD),
                       mlp2_w[l], mlp2_b[l].reshape(1, E))
    tgt = st[CT - 1, SPLIT:ROWS]
    out = _dec_call(tgt, dec1_w, dec1_b.reshape(1, MLPD),
                    dec2_w, dec2_b.reshape(1, 2))
    return out.reshape(1, ROWS - SPLIT, 2)


# speedup vs baseline: 11.4881x; 1.1526x over previous
import jax
import jax.numpy as jnp
from jax.experimental import pallas as pl

E = 192
NHEAD = 4
HD = 48
MLPD = 384
TOPK = 16
SPLIT = 256
ROWS = 512
COLS = 16
CT = 17
IDXH = 4
IDXD = 8
LAT = 96
RB = 16
CHUNK = RB * CT
HRB = 8
HCH = HRB * CT
NCH = ROWS // RB
PADR = ROWS


def _dt(a, b, prec=None):
    return jax.lax.dot_general(a, b, (((1,), (1,)), ((), ())),
                               precision=prec, preferred_element_type=jnp.float32)


def _dot(a, b, prec=None):
    return jax.lax.dot_general(a, b, (((1,), (0,)), ((), ())),
                               precision=prec, preferred_element_type=jnp.float32)


def _split(a):
    hi = a.astype(jnp.bfloat16).astype(jnp.float32)
    return hi, a - hi


def _ln(x, g, b):
    m = jnp.mean(x, axis=-1, keepdims=True)
    v = jnp.mean((x - m) ** 2, axis=-1, keepdims=True)
    return (x - m) / jnp.sqrt(v + 1e-5) * g + b


def _gelu(x):
    return 0.5 * x * (1.0 + jax.lax.erf(x / jnp.sqrt(jnp.float32(2.0))))


def _embed_body(x_ref, w_ref, b_ref, out_ref):
    out_ref[0] = x_ref[0] * w_ref[0] + b_ref[0]


def _embed_call(xcols, w, b):
    return pl.pallas_call(
        _embed_body,
        grid=(CT,),
        in_specs=[pl.BlockSpec((1, PADR, 1), lambda f: (f, 0, 0)),
                  pl.BlockSpec((1, 1, E), lambda f: (f, 0, 0)),
                  pl.BlockSpec((1, 1, E), lambda f: (f, 0, 0))],
        out_specs=pl.BlockSpec((1, PADR, E), lambda f: (f, 0, 0)),
        out_shape=jax.ShapeDtypeStruct((CT, PADR, E), jnp.float32),
    )(xcols, w, b)


def _fa_body(tok_ref, wq_ref, wk_ref, wv_ref, bq_ref, bk_ref, bv_ref,
             wo_ref, bo_ref, g_ref, b_ref, rc_ref, rr_ref, out_ref):
    blk = rc_ref[...] == rr_ref[...]
    li = jax.lax.broadcasted_iota(jnp.int32, (1, E), 1)
    for half in range(RB // HRB):
        x = tok_ref[:, half * HRB:(half + 1) * HRB, :].reshape(HCH, E)
        q = _dt(x, wq_ref[...]) + bq_ref[...]
        k = _dt(x, wk_ref[...]) + bk_ref[...]
        v = _dt(x, wv_ref[...]) + bv_ref[...]
        acc = jnp.zeros((HCH, E), jnp.float32)
        for h in range(NHEAD):
            hm = ((li >= h * HD) & (li < (h + 1) * HD)).astype(jnp.float32)
            s = _dt(q * hm, k) / jnp.sqrt(jnp.float32(HD))
            p = jnp.where(blk, jnp.exp(s), 0.0)
            rz = 1.0 / jnp.sum(p, axis=-1, keepdims=True)
            acc = acc + _dot(p, v * hm) * rz
        o = _dt(acc, wo_ref[...]) + bo_ref[...] + x
        out_ref[:, half * HRB:(half + 1) * HRB, :] = (
            _ln(o, g_ref[...], b_ref[...]).reshape(CT, HRB, E))


def _fa_call(st, wq, wk, wv, bq, bk, bv, wo, bo, g, b, rc, rr):
    def full(a):
        shp = a.shape
        return pl.BlockSpec(shp, lambda i: tuple(0 for _ in shp))
    consts = (wq, wk, wv, bq, bk, bv, wo, bo, g, b, rc, rr)
    return pl.pallas_call(
        _fa_body,
        grid=(NCH,),
        in_specs=[pl.BlockSpec((CT, RB, E), lambda i: (0, i, 0))] + [full(a) for a in consts],
        out_specs=pl.BlockSpec((CT, RB, E), lambda i: (0, i, 0)),
        out_shape=jax.ShapeDtypeStruct((CT, PADR, E), jnp.float32),
    )(st, *consts)


def _bcd_body(st_ref, qw_ref, kw_ref, ow_ref, mqw_ref, down_ref, upk_ref,
              upv_ref, mout_ref, n2g_ref, n2b_ref, n3g_ref, n3b_ref,
              m1w_ref, m1b_ref, m2w_ref, m2b_ref, out_ref):
    x = st_ref[0][:ROWS] if PADR > ROWS else st_ref[0]
    keys = x[:SPLIT]

    qi = _dt(x, qw_ref[...])
    ki = _dt(keys, kw_ref[...])
    li = jax.lax.broadcasted_iota(jnp.int32, (1, IDXH * IDXD), 1)
    hms = [((li >= h * IDXD) & (li < (h + 1) * IDXD)).astype(jnp.float32)
           for h in range(IDXH)]
    k_abs = jnp.abs(ki)
    ks = [(jnp.max(k_abs * hms[h], keepdims=True) + 1e-6) / 127.0
          for h in range(IDXH)]
    k_scale_vec = sum(ks[h] * hms[h] for h in range(IDXH))
    k_q = jnp.clip(jnp.round(ki / k_scale_vec), -127.0, 127.0)

    halves = []
    for off in (0, SPLIT):
        qh = qi[off:off + SPLIT]
        q_abs = jnp.abs(qh)
        qs = [(jnp.max(q_abs * hms[h], keepdims=True) + 1e-6) / 127.0
              for h in range(IDXH)]
        q_scale_vec = sum(qs[h] * hms[h] for h in range(IDXH))
        q_q = jnp.clip(jnp.round(qh / q_scale_vec), -127.0, 127.0)
        red = jnp.zeros((SPLIT, SPLIT), jnp.float32)
        for h in range(IDXH):
            s = _dt(q_q * hms[h], k_q)
            red = red + (qs[h] * ks[h] * ow_ref[0:1, h:h + 1]) * jax.nn.relu(s)
        halves.append(red)
    red = jnp.concatenate(halves, axis=0)

    u = jax.lax.bitcast_convert_type(red, jnp.int32)
    key = jnp.where(u >= 0, u, jnp.int32(-2147483648) - u)
    iot = jax.lax.broadcasted_iota(jnp.int32, (ROWS, SPLIT), 1)
    cur = (key & jnp.int32(-256)) | (jnp.int32(255) - iot)
    mask = jnp.zeros((ROWS, SPLIT), jnp.float32)
    for _ in range(TOPK):
        m = jnp.max(cur, axis=-1, keepdims=True)
        oh = cur == m
        mask = jnp.where(oh, 1.0, mask)
        cur = jnp.where(oh, jnp.int32(-2147483648), cur)

    c = _dt(keys, down_ref[...])
    k_all = _dt(c, upk_ref[...])
    v_all = _dt(c, upv_ref[...])
    qm = _dt(x, mqw_ref[...])
    liE = jax.lax.broadcasted_iota(jnp.int32, (1, E), 1)
    o = jnp.zeros((ROWS, E), jnp.float32)
    sel = mask > 0.5
    qm_hi, qm_lo = _split(qm)
    ka_hi, ka_lo = _split(k_all)
    va_hi, va_lo = _split(v_all)
    for h in range(NHEAD):
        hm = ((liE >= h * HD) & (liE < (h + 1) * HD)).astype(jnp.float32)
        a_hi = qm_hi * hm
        a_lo = qm_lo * hm
        s = (_dt(a_hi, ka_hi) + (_dt(a_lo, ka_hi) + _dt(a_hi, ka_lo)))
        s = s / jnp.sqrt(jnp.float32(HD))
        p = jnp.where(sel, jnp.exp(s), 0.0)
        rz = 1.0 / jnp.sum(p, axis=-1, keepdims=True)
        p_hi, p_lo = _split(p)
        vm_hi = va_hi * hm
        vm_lo = va_lo * hm
        o = o + (_dot(p_hi, vm_hi) + (_dot(p_lo, vm_hi) + _dot(p_hi, vm_lo))) * rz
    attn = _dt(o, mout_ref[...])
    src2 = _ln(x + attn, n2g_ref[...], n2b_ref[...])

    h1 = _gelu(_dt(src2, m1w_ref[...]) + m1b_ref[...])
    src3 = _dt(h1, m2w_ref[...]) + m2b_ref[...] + src2
    out_ref[0, :ROWS] = _ln(src3, n3g_ref[...], n3b_ref[...])
    if PADR > ROWS:
        out_ref[0, ROWS:] = jnp.zeros((PADR - ROWS, E), jnp.float32)


def _bcd_call(st, qw, kw, ow, mqw, down, upk, upv, mout,
              n2g, n2b, n3g, n3b, m1w, m1b, m2w, m2b):
    def full(a):
        shp = a.shape
        return pl.BlockSpec(shp, lambda i: tuple(0 for _ in shp))
    consts = (qw, kw, ow, mqw, down, upk, upv, mout,
              n2g, n2b, n3g, n3b, m1w, m1b, m2w, m2b)
    return pl.pallas_call(
        _bcd_body,
        grid=(CT,),
        in_specs=[pl.BlockSpec((1, PADR, E), lambda f: (f, 0, 0))] + [full(a) for a in consts],
        out_specs=pl.BlockSpec((1, PADR, E), lambda f: (f, 0, 0)),
        out_shape=jax.ShapeDtypeStruct((CT, PADR, E), jnp.float32),
    )(st, *consts)


def _dec_body(x_ref, w1_ref, b1_ref, w2_ref, b2_ref, out_ref):
    h = _gelu(_dt(x_ref[...], w1_ref[...]) + b1_ref[...])
    out_ref[...] = _dt(h, w2_ref[...]) + b2_ref[...]


def _dec_call(x, w1, b1, w2, b2):
    return pl.pallas_call(
        _dec_body,
        out_shape=jax.ShapeDtypeStruct((ROWS - SPLIT, 2), jnp.float32),
    )(x, w1, b1, w2, b2)


def kernel(x_src, y_src, fa_in_w, fa_in_b, fa_out_w, fa_out_b,
           idx_qw, idx_kw, idx_ow, mla_qw, mla_down, mla_up, mla_out,
           n1_g, n1_b, n2_g, n2_b, n3_g, n3_b,
           mlp1_w, mlp1_b, mlp2_w, mlp2_b,
           fe_w, fe_b, te_w, te_b, dec1_w, dec1_b, dec2_w, dec2_b):
    f32 = jnp.float32
    y_full = jnp.concatenate([y_src[0], jnp.zeros((ROWS - SPLIT, 1), f32)], axis=0)
    xcols = jnp.pad(jnp.concatenate([x_src[0], y_full], axis=1).T,
                    ((0, 0), (0, PADR - ROWS))).reshape(CT, PADR, 1)
    w_emb = jnp.concatenate([jnp.tile(fe_w.reshape(1, E), (COLS, 1)),
                             te_w.reshape(1, E)], axis=0).reshape(CT, 1, E)
    b_emb = jnp.concatenate([jnp.tile(fe_b.reshape(1, E), (COLS, 1)),
                             te_b.reshape(1, E)], axis=0).reshape(CT, 1, E)
    st = _embed_call(xcols, w_emb, b_emb)
    rc = (jnp.arange(HCH, dtype=jnp.int32) % HRB).reshape(HCH, 1)
    rr = rc.reshape(1, HCH)
    for l in range(2):
        st = _fa_call(st,
                      fa_in_w[l, 0:E], fa_in_w[l, E:2 * E], fa_in_w[l, 2 * E:3 * E],
                      fa_in_b[l, 0:E].reshape(1, E), fa_in_b[l, E:2 * E].reshape(1, E),
                      fa_in_b[l, 2 * E:3 * E].reshape(1, E),
                      fa_out_w[l], fa_out_b[l].reshape(1, E),
                      n1_g[l].reshape(1, E), n1_b[l].reshape(1, E), rc, rr)
        st = _bcd_call(st, idx_qw[l], idx_kw[l], idx_ow[l].reshape(1, IDXH),
                       mla_qw[l], mla_down[l], mla_up[l][:E], mla_up[l][E:],
                       mla_out[l],
                       n2_g[l].reshape(1, E), n2_b[l].reshape(1, E),
                       n3_g[l].reshape(1, E), n3_b[l].reshape(1, E),
                       mlp1_w[l], mlp1_b[l].reshape(1, MLPD),
                       mlp2_w[l], mlp2_b[l].reshape(1, E))
    tgt = st[CT - 1, SPLIT:ROWS]
    out = _dec_call(tgt, dec1_w, dec1_b.reshape(1, MLPD),
                    dec2_w, dec2_b.reshape(1, 2))
    return out.reshape(1, ROWS - SPLIT, 2)


# speedup vs baseline: 12.8368x; 1.1174x over previous
import jax
import jax.numpy as jnp
from jax.experimental import pallas as pl

E = 192
NHEAD = 4
HD = 48
MLPD = 384
TOPK = 16
SPLIT = 256
ROWS = 512
COLS = 16
CT = 17
IDXH = 4
IDXD = 8
LAT = 96
RB = 16
CHUNK = RB * CT
HRB = 8
HCH = HRB * CT
NCH = ROWS // RB
PADR = ROWS


def _dt(a, b, prec=None):
    return jax.lax.dot_general(a, b, (((1,), (1,)), ((), ())),
                               precision=prec, preferred_element_type=jnp.float32)


def _dot(a, b, prec=None):
    return jax.lax.dot_general(a, b, (((1,), (0,)), ((), ())),
                               precision=prec, preferred_element_type=jnp.float32)


def _split(a):
    hi = a.astype(jnp.bfloat16).astype(jnp.float32)
    return hi, a - hi


def _ln(x, g, b):
    m = jnp.mean(x, axis=-1, keepdims=True)
    v = jnp.mean((x - m) ** 2, axis=-1, keepdims=True)
    return (x - m) / jnp.sqrt(v + 1e-5) * g + b


def _gelu(x):
    return 0.5 * x * (1.0 + jax.lax.erf(x / jnp.sqrt(jnp.float32(2.0))))


def _embed_body(x_ref, w_ref, b_ref, out_ref):
    out_ref[0] = x_ref[0] * w_ref[0] + b_ref[0]


def _embed_call(xcols, w, b):
    return pl.pallas_call(
        _embed_body,
        grid=(CT,),
        in_specs=[pl.BlockSpec((1, PADR, 1), lambda f: (f, 0, 0)),
                  pl.BlockSpec((1, 1, E), lambda f: (f, 0, 0)),
                  pl.BlockSpec((1, 1, E), lambda f: (f, 0, 0))],
        out_specs=pl.BlockSpec((1, PADR, E), lambda f: (f, 0, 0)),
        out_shape=jax.ShapeDtypeStruct((CT, PADR, E), jnp.float32),
    )(xcols, w, b)


def _fa_body(tok_ref, wq_ref, wk_ref, wv_ref, bq_ref, bk_ref, bv_ref,
             wo_ref, bo_ref, g_ref, b_ref, rc_ref, rr_ref, out_ref):
    blk = rc_ref[...] == rr_ref[...]
    li = jax.lax.broadcasted_iota(jnp.int32, (1, E), 1)
    for half in range(RB // HRB):
        x = tok_ref[:, half * HRB:(half + 1) * HRB, :].reshape(HCH, E)
        q = _dt(x, wq_ref[...]) + bq_ref[...]
        k = _dt(x, wk_ref[...]) + bk_ref[...]
        v = _dt(x, wv_ref[...]) + bv_ref[...]
        acc = jnp.zeros((HCH, E), jnp.float32)
        for h in range(NHEAD):
            hm = ((li >= h * HD) & (li < (h + 1) * HD)).astype(jnp.float32)
            s = _dt(q * hm, k) / jnp.sqrt(jnp.float32(HD))
            p = jnp.where(blk, jnp.exp(s), 0.0)
            rz = 1.0 / jnp.sum(p, axis=-1, keepdims=True)
            acc = acc + _dot(p, v * hm) * rz
        o = _dt(acc, wo_ref[...]) + bo_ref[...] + x
        out_ref[:, half * HRB:(half + 1) * HRB, :] = (
            _ln(o, g_ref[...], b_ref[...]).reshape(CT, HRB, E))


def _fa_call(st, wq, wk, wv, bq, bk, bv, wo, bo, g, b, rc, rr):
    def full(a):
        shp = a.shape
        return pl.BlockSpec(shp, lambda i: tuple(0 for _ in shp))
    consts = (wq, wk, wv, bq, bk, bv, wo, bo, g, b, rc, rr)
    return pl.pallas_call(
        _fa_body,
        grid=(NCH,),
        in_specs=[pl.BlockSpec((CT, RB, E), lambda i: (0, i, 0))] + [full(a) for a in consts],
        out_specs=pl.BlockSpec((CT, RB, E), lambda i: (0, i, 0)),
        out_shape=jax.ShapeDtypeStruct((CT, PADR, E), jnp.float32),
    )(st, *consts)


def _bcd_body(st_ref, qw_ref, kw_ref, ow_ref, mqw_ref, down_ref, upk_ref,
              upv_ref, mout_ref, n2g_ref, n2b_ref, n3g_ref, n3b_ref,
              m1w_ref, m1b_ref, m2w_ref, m2b_ref, out_ref):
    x = st_ref[0][:ROWS] if PADR > ROWS else st_ref[0]
    keys = x[:SPLIT]

    c = _dt(keys, down_ref[...])
    k_all = _dt(c, upk_ref[...])
    v_all = _dt(c, upv_ref[...])
    qm = _dt(x, mqw_ref[...])
    qm_hi, qm_lo = _split(qm)
    ka_hi, ka_lo = _split(k_all)
    va_hi, va_lo = _split(v_all)

    qi = _dt(x, qw_ref[...])
    ki = _dt(keys, kw_ref[...])
    li = jax.lax.broadcasted_iota(jnp.int32, (1, IDXH * IDXD), 1)
    hms = [((li >= h * IDXD) & (li < (h + 1) * IDXD)).astype(jnp.float32)
           for h in range(IDXH)]
    k_abs = jnp.abs(ki)
    ks = [(jnp.max(k_abs * hms[h], keepdims=True) + 1e-6) / 127.0
          for h in range(IDXH)]
    k_scale_vec = sum(ks[h] * hms[h] for h in range(IDXH))
    k_q = jnp.clip(jnp.round(ki / k_scale_vec), -127.0, 127.0)

    halves = []
    for off in (0, SPLIT):
        qh = qi[off:off + SPLIT]
        q_abs = jnp.abs(qh)
        qs = [(jnp.max(q_abs * hms[h], keepdims=True) + 1e-6) / 127.0
              for h in range(IDXH)]
        q_scale_vec = sum(qs[h] * hms[h] for h in range(IDXH))
        q_q = jnp.clip(jnp.round(qh / q_scale_vec), -127.0, 127.0)
        red = jnp.zeros((SPLIT, SPLIT), jnp.float32)
        for h in range(IDXH):
            s = _dt(q_q * hms[h], k_q)
            red = red + (qs[h] * ks[h] * ow_ref[0:1, h:h + 1]) * jax.nn.relu(s)
        halves.append(red)
    red = jnp.concatenate(halves, axis=0)

    u = jax.lax.bitcast_convert_type(red, jnp.int32)
    key = jnp.where(u >= 0, u, jnp.int32(-2147483648) - u)
    iot = jax.lax.broadcasted_iota(jnp.int32, (ROWS, SPLIT), 1)
    cur = ((key & jnp.int32(-256)) | (jnp.int32(255) - iot)).T
    for _ in range(TOPK):
        m = jnp.max(cur, axis=0, keepdims=True)
        cur = jnp.where(cur == m, jnp.int32(-2147483648), cur)
    sel = (cur == jnp.int32(-2147483648)).T

    liE = jax.lax.broadcasted_iota(jnp.int32, (1, E), 1)
    o = jnp.zeros((ROWS, E), jnp.float32)
    for h in range(NHEAD):
        hm = ((liE >= h * HD) & (liE < (h + 1) * HD)).astype(jnp.float32)
        a_hi = qm_hi * hm
        a_lo = qm_lo * hm
        s = (_dt(a_hi, ka_hi) + (_dt(a_lo, ka_hi) + _dt(a_hi, ka_lo)))
        s = s / jnp.sqrt(jnp.float32(HD))
        p = jnp.where(sel, jnp.exp(s), 0.0)
        rz = 1.0 / jnp.sum(p, axis=-1, keepdims=True)
        p_hi, p_lo = _split(p)
        vm_hi = va_hi * hm
        vm_lo = va_lo * hm
        o = o + (_dot(p_hi, vm_hi) + (_dot(p_lo, vm_hi) + _dot(p_hi, vm_lo))) * rz
    attn = _dt(o, mout_ref[...])
    src2 = _ln(x + attn, n2g_ref[...], n2b_ref[...])

    h1 = _gelu(_dt(src2, m1w_ref[...]) + m1b_ref[...])
    src3 = _dt(h1, m2w_ref[...]) + m2b_ref[...] + src2
    out_ref[0, :ROWS] = _ln(src3, n3g_ref[...], n3b_ref[...])
    if PADR > ROWS:
        out_ref[0, ROWS:] = jnp.zeros((PADR - ROWS, E), jnp.float32)


def _bcd_call(st, qw, kw, ow, mqw, down, upk, upv, mout,
              n2g, n2b, n3g, n3b, m1w, m1b, m2w, m2b):
    def full(a):
        shp = a.shape
        return pl.BlockSpec(shp, lambda i: tuple(0 for _ in shp))
    consts = (qw, kw, ow, mqw, down, upk, upv, mout,
              n2g, n2b, n3g, n3b, m1w, m1b, m2w, m2b)
    return pl.pallas_call(
        _bcd_body,
        grid=(CT,),
        in_specs=[pl.BlockSpec((1, PADR, E), lambda f: (f, 0, 0))] + [full(a) for a in consts],
        out_specs=pl.BlockSpec((1, PADR, E), lambda f: (f, 0, 0)),
        out_shape=jax.ShapeDtypeStruct((CT, PADR, E), jnp.float32),
    )(st, *consts)


def _dec_body(x_ref, w1_ref, b1_ref, w2_ref, b2_ref, out_ref):
    h = _gelu(_dt(x_ref[...], w1_ref[...]) + b1_ref[...])
    out_ref[...] = _dt(h, w2_ref[...]) + b2_ref[...]


def _dec_call(x, w1, b1, w2, b2):
    return pl.pallas_call(
        _dec_body,
        out_shape=jax.ShapeDtypeStruct((ROWS - SPLIT, 2), jnp.float32),
    )(x, w1, b1, w2, b2)


def kernel(x_src, y_src, fa_in_w, fa_in_b, fa_out_w, fa_out_b,
           idx_qw, idx_kw, idx_ow, mla_qw, mla_down, mla_up, mla_out,
           n1_g, n1_b, n2_g, n2_b, n3_g, n3_b,
           mlp1_w, mlp1_b, mlp2_w, mlp2_b,
           fe_w, fe_b, te_w, te_b, dec1_w, dec1_b, dec2_w, dec2_b):
    f32 = jnp.float32
    y_full = jnp.concatenate([y_src[0], jnp.zeros((ROWS - SPLIT, 1), f32)], axis=0)
    xcols = jnp.pad(jnp.concatenate([x_src[0], y_full], axis=1).T,
                    ((0, 0), (0, PADR - ROWS))).reshape(CT, PADR, 1)
    w_emb = jnp.concatenate([jnp.tile(fe_w.reshape(1, E), (COLS, 1)),
                             te_w.reshape(1, E)], axis=0).reshape(CT, 1, E)
    b_emb = jnp.concatenate([jnp.tile(fe_b.reshape(1, E), (COLS, 1)),
                             te_b.reshape(1, E)], axis=0).reshape(CT, 1, E)
    st = _embed_call(xcols, w_emb, b_emb)
    rc = (jnp.arange(HCH, dtype=jnp.int32) % HRB).reshape(HCH, 1)
    rr = rc.reshape(1, HCH)
    for l in range(2):
        st = _fa_call(st,
                      fa_in_w[l, 0:E], fa_in_w[l, E:2 * E], fa_in_w[l, 2 * E:3 * E],
                      fa_in_b[l, 0:E].reshape(1, E), fa_in_b[l, E:2 * E].reshape(1, E),
                      fa_in_b[l, 2 * E:3 * E].reshape(1, E),
                      fa_out_w[l], fa_out_b[l].reshape(1, E),
                      n1_g[l].reshape(1, E), n1_b[l].reshape(1, E), rc, rr)
        st = _bcd_call(st, idx_qw[l], idx_kw[l], idx_ow[l].reshape(1, IDXH),
                       mla_qw[l], mla_down[l], mla_up[l][:E], mla_up[l][E:],
                       mla_out[l],
                       n2_g[l].reshape(1, E), n2_b[l].reshape(1, E),
                       n3_g[l].reshape(1, E), n3_b[l].reshape(1, E),
                       mlp1_w[l], mlp1_b[l].reshape(1, MLPD),
                       mlp2_w[l], mlp2_b[l].reshape(1, E))
    tgt = st[CT - 1, SPLIT:ROWS]
    out = _dec_call(tgt, dec1_w, dec1_b.reshape(1, MLPD),
                    dec2_w, dec2_b.reshape(1, 2))
    return out.reshape(1, ROWS - SPLIT, 2)


# speedup vs baseline: 13.6623x; 1.0643x over previous
import functools

import jax
import jax.numpy as jnp
from jax.experimental import pallas as pl

E = 192
NHEAD = 4
HD = 48
MLPD = 384
TOPK = 16
SPLIT = 256
ROWS = 512
COLS = 16
CT = 17
IDXH = 4
IDXD = 8
LAT = 96
RB = 16
CHUNK = RB * CT
HRB = 8
HCH = HRB * CT
NCH = ROWS // RB
PADR = ROWS


def _dt(a, b, prec=None):
    return jax.lax.dot_general(a, b, (((1,), (1,)), ((), ())),
                               precision=prec, preferred_element_type=jnp.float32)


def _dot(a, b, prec=None):
    return jax.lax.dot_general(a, b, (((1,), (0,)), ((), ())),
                               precision=prec, preferred_element_type=jnp.float32)


def _split(a):
    hi = a.astype(jnp.bfloat16).astype(jnp.float32)
    return hi, a - hi


def _ln(x, g, b):
    m = jnp.mean(x, axis=-1, keepdims=True)
    v = jnp.mean((x - m) ** 2, axis=-1, keepdims=True)
    inv = 1.0 / jnp.sqrt(v + 1e-5)
    return (x - m) * inv * g + b


def _gelu(x):
    return 0.5 * x * (1.0 + jax.lax.erf(x / jnp.sqrt(jnp.float32(2.0))))


def _embed_body(x_ref, w_ref, b_ref, out_ref):
    out_ref[0] = x_ref[0] * w_ref[0] + b_ref[0]


def _embed_call(xcols, w, b):
    return pl.pallas_call(
        _embed_body,
        grid=(CT,),
        in_specs=[pl.BlockSpec((1, PADR, 1), lambda f: (f, 0, 0)),
                  pl.BlockSpec((1, 1, E), lambda f: (f, 0, 0)),
                  pl.BlockSpec((1, 1, E), lambda f: (f, 0, 0))],
        out_specs=pl.BlockSpec((1, PADR, E), lambda f: (f, 0, 0)),
        out_shape=jax.ShapeDtypeStruct((CT, PADR, E), jnp.float32),
    )(xcols, w, b)


def _fa_body(tok_ref, wq_ref, wk_ref, wv_ref, bq_ref, bk_ref, bv_ref,
             wo_ref, bo_ref, g_ref, b_ref, rc_ref, rr_ref, out_ref):
    blk = rc_ref[...] == rr_ref[...]
    li = jax.lax.broadcasted_iota(jnp.int32, (1, E), 1)
    for half in range(RB // HRB):
        x = tok_ref[:, half * HRB:(half + 1) * HRB, :].reshape(HCH, E)
        q = _dt(x, wq_ref[...]) + bq_ref[...]
        k = _dt(x, wk_ref[...]) + bk_ref[...]
        v = _dt(x, wv_ref[...]) + bv_ref[...]
        acc = jnp.zeros((HCH, E), jnp.float32)
        for h in range(NHEAD):
            hm = ((li >= h * HD) & (li < (h + 1) * HD)).astype(jnp.float32)
            s = _dt(q * hm, k) * jnp.float32(1.0 / 48.0 ** 0.5)
            p = jnp.where(blk, jnp.exp(s), 0.0)
            rz = 1.0 / jnp.sum(p, axis=-1, keepdims=True)
            acc = acc + _dot(p, v * hm) * rz
        o = _dt(acc, wo_ref[...]) + bo_ref[...] + x
        out_ref[:, half * HRB:(half + 1) * HRB, :] = (
            _ln(o, g_ref[...], b_ref[...]).reshape(CT, HRB, E))


def _fa_call(st, wq, wk, wv, bq, bk, bv, wo, bo, g, b, rc, rr):
    def full(a):
        shp = a.shape
        return pl.BlockSpec(shp, lambda i: tuple(0 for _ in shp))
    consts = (wq, wk, wv, bq, bk, bv, wo, bo, g, b, rc, rr)
    return pl.pallas_call(
        _fa_body,
        grid=(NCH,),
        in_specs=[pl.BlockSpec((CT, RB, E), lambda i: (0, i, 0))] + [full(a) for a in consts],
        out_specs=pl.BlockSpec((CT, RB, E), lambda i: (0, i, 0)),
        out_shape=jax.ShapeDtypeStruct((CT, PADR, E), jnp.float32),
    )(st, *consts)


def _bcd_body(st_ref, qw_ref, kw_ref, ow_ref, mqw_ref, down_ref, upk_ref,
              upv_ref, mout_ref, n2g_ref, n2b_ref, n3g_ref, n3b_ref,
              m1w_ref, m1b_ref, m2w_ref, m2b_ref, out_ref, *, precise):
    x = st_ref[0][:ROWS] if PADR > ROWS else st_ref[0]
    keys = x[:SPLIT]

    c = _dt(keys, down_ref[...])
    k_all = _dt(c, upk_ref[...])
    v_all = _dt(c, upv_ref[...])
    qm = _dt(x, mqw_ref[...])
    if precise:
        qm_hi, qm_lo = _split(qm)
        ka_hi, ka_lo = _split(k_all)
        va_hi, va_lo = _split(v_all)

    qi = _dt(x, qw_ref[...])
    ki = _dt(keys, kw_ref[...])
    li = jax.lax.broadcasted_iota(jnp.int32, (1, IDXH * IDXD), 1)
    hms = [((li >= h * IDXD) & (li < (h + 1) * IDXD)).astype(jnp.float32)
           for h in range(IDXH)]
    k_cmax = jnp.max(jnp.abs(ki), axis=0, keepdims=True)
    ks = [(jnp.max(k_cmax * hms[h], keepdims=True) + 1e-6) / 127.0
          for h in range(IDXH)]
    k_rscale = sum((1.0 / ks[h]) * hms[h] for h in range(IDXH))
    k_q = jnp.clip(jnp.round(ki * k_rscale), -127.0, 127.0)

    halves = []
    for off in (0, SPLIT):
        qh = qi[off:off + SPLIT]
        q_cmax = jnp.max(jnp.abs(qh), axis=0, keepdims=True)
        qs = [(jnp.max(q_cmax * hms[h], keepdims=True) + 1e-6) / 127.0
              for h in range(IDXH)]
        q_rscale = sum((1.0 / qs[h]) * hms[h] for h in range(IDXH))
        q_q = jnp.clip(jnp.round(qh * q_rscale), -127.0, 127.0)
        red = jnp.zeros((SPLIT, SPLIT), jnp.float32)
        for h in range(IDXH):
            s = _dt(q_q * hms[h], k_q)
            red = red + (qs[h] * ks[h] * ow_ref[0:1, h:h + 1]) * jax.nn.relu(s)
        halves.append(red)
    red = jnp.concatenate(halves, axis=0)

    u = jax.lax.bitcast_convert_type(red, jnp.int32)
    key = jnp.where(u >= 0, u, jnp.int32(-2147483648) - u)
    iot = jax.lax.broadcasted_iota(jnp.int32, (ROWS, SPLIT), 1)
    cur = ((key & jnp.int32(-256)) | (jnp.int32(255) - iot)).T
    for _ in range(TOPK):
        m = jnp.max(cur, axis=0, keepdims=True)
        cur = jnp.where(cur == m, jnp.int32(-2147483648), cur)
    sel = (cur == jnp.int32(-2147483648)).T

    liE = jax.lax.broadcasted_iota(jnp.int32, (1, E), 1)
    o = jnp.zeros((ROWS, E), jnp.float32)
    for h in range(NHEAD):
        hm = ((liE >= h * HD) & (liE < (h + 1) * HD)).astype(jnp.float32)
        if precise:
            a_hi = qm_hi * hm
            a_lo = qm_lo * hm
            s = (_dt(a_hi, ka_hi) + (_dt(a_lo, ka_hi) + _dt(a_hi, ka_lo)))
        else:
            s = _dt(qm * hm, k_all)
        s = s * jnp.float32(1.0 / 48.0 ** 0.5)
        p = jnp.where(sel, jnp.exp(s), 0.0)
        rz = 1.0 / jnp.sum(p, axis=-1, keepdims=True)
        if precise:
            p_hi, p_lo = _split(p)
            vm_hi = va_hi * hm
            vm_lo = va_lo * hm
            o = o + (_dot(p_hi, vm_hi) + (_dot(p_lo, vm_hi) + _dot(p_hi, vm_lo))) * rz
        else:
            o = o + _dot(p, v_all * hm) * rz
    attn = _dt(o, mout_ref[...])
    src2 = _ln(x + attn, n2g_ref[...], n2b_ref[...])

    h1 = _gelu(_dt(src2, m1w_ref[...]) + m1b_ref[...])
    src3 = _dt(h1, m2w_ref[...]) + m2b_ref[...] + src2
    out_ref[0, :ROWS] = _ln(src3, n3g_ref[...], n3b_ref[...])
    if PADR > ROWS:
        out_ref[0, ROWS:] = jnp.zeros((PADR - ROWS, E), jnp.float32)


def _bcd_call(st, qw, kw, ow, mqw, down, upk, upv, mout,
              n2g, n2b, n3g, n3b, m1w, m1b, m2w, m2b, precise):
    def full(a):
        shp = a.shape
        return pl.BlockSpec(shp, lambda i: tuple(0 for _ in shp))
    consts = (qw, kw, ow, mqw, down, upk, upv, mout,
              n2g, n2b, n3g, n3b, m1w, m1b, m2w, m2b)
    return pl.pallas_call(
        functools.partial(_bcd_body, precise=precise),
        grid=(CT,),
        in_specs=[pl.BlockSpec((1, PADR, E), lambda f: (f, 0, 0))] + [full(a) for a in consts],
        out_specs=pl.BlockSpec((1, PADR, E), lambda f: (f, 0, 0)),
        out_shape=jax.ShapeDtypeStruct((CT, PADR, E), jnp.float32),
    )(st, *consts)


def _dec_body(x_ref, w1_ref, b1_ref, w2_ref, b2_ref, out_ref):
    h = _gelu(_dt(x_ref[...], w1_ref[...]) + b1_ref[...])
    out_ref[...] = _dt(h, w2_ref[...]) + b2_ref[...]


def _dec_call(x, w1, b1, w2, b2):
    return pl.pallas_call(
        _dec_body,
        out_shape=jax.ShapeDtypeStruct((ROWS - SPLIT, 2), jnp.float32),
    )(x, w1, b1, w2, b2)


def kernel(x_src, y_src, fa_in_w, fa_in_b, fa_out_w, fa_out_b,
           idx_qw, idx_kw, idx_ow, mla_qw, mla_down, mla_up, mla_out,
           n1_g, n1_b, n2_g, n2_b, n3_g, n3_b,
           mlp1_w, mlp1_b, mlp2_w, mlp2_b,
           fe_w, fe_b, te_w, te_b, dec1_w, dec1_b, dec2_w, dec2_b):
    f32 = jnp.float32
    y_full = jnp.concatenate([y_src[0], jnp.zeros((ROWS - SPLIT, 1), f32)], axis=0)
    xcols = jnp.pad(jnp.concatenate([x_src[0], y_full], axis=1).T,
                    ((0, 0), (0, PADR - ROWS))).reshape(CT, PADR, 1)
    w_emb = jnp.concatenate([jnp.tile(fe_w.reshape(1, E), (COLS, 1)),
                             te_w.reshape(1, E)], axis=0).reshape(CT, 1, E)
    b_emb = jnp.concatenate([jnp.tile(fe_b.reshape(1, E), (COLS, 1)),
                             te_b.reshape(1, E)], axis=0).reshape(CT, 1, E)
    st = _embed_call(xcols, w_emb, b_emb)
    rc = (jnp.arange(HCH, dtype=jnp.int32) % HRB).reshape(HCH, 1)
    rr = rc.reshape(1, HCH)
    for l in range(2):
        st = _fa_call(st,
                      fa_in_w[l, 0:E], fa_in_w[l, E:2 * E], fa_in_w[l, 2 * E:3 * E],
                      fa_in_b[l, 0:E].reshape(1, E), fa_in_b[l, E:2 * E].reshape(1, E),
                      fa_in_b[l, 2 * E:3 * E].reshape(1, E),
                      fa_out_w[l], fa_out_b[l].reshape(1, E),
                      n1_g[l].reshape(1, E), n1_b[l].reshape(1, E), rc, rr)
        st = _bcd_call(st, idx_qw[l], idx_kw[l], idx_ow[l].reshape(1, IDXH),
                       mla_qw[l], mla_down[l], mla_up[l][:E], mla_up[l][E:],
                       mla_out[l],
                       n2_g[l].reshape(1, E), n2_b[l].reshape(1, E),
                       n3_g[l].reshape(1, E), n3_b[l].reshape(1, E),
                       mlp1_w[l], mlp1_b[l].reshape(1, MLPD),
                       mlp2_w[l], mlp2_b[l].reshape(1, E), precise=(l == 0))
    tgt = st[CT - 1, SPLIT:ROWS]
    out = _dec_call(tgt, dec1_w, dec1_b.reshape(1, MLPD),
                    dec2_w, dec2_b.reshape(1, 2))
    return out.reshape(1, ROWS - SPLIT, 2)


# speedup vs baseline: 14.4900x; 1.0606x over previous
import functools

import jax
import jax.numpy as jnp
from jax.experimental import pallas as pl

E = 192
NHEAD = 4
HD = 48
MLPD = 384
TOPK = 16
SPLIT = 256
ROWS = 512
COLS = 16
CT = 17
IDXH = 4
IDXD = 8
LAT = 96
RB = 16
CHUNK = RB * CT
HRB = 8
HCH = HRB * CT
NCH = ROWS // RB
PADR = ROWS


def _dt(a, b, prec=None):
    return jax.lax.dot_general(a, b, (((1,), (1,)), ((), ())),
                               precision=prec, preferred_element_type=jnp.float32)


def _dot(a, b, prec=None):
    return jax.lax.dot_general(a, b, (((1,), (0,)), ((), ())),
                               precision=prec, preferred_element_type=jnp.float32)


def _split(a):
    hi = a.astype(jnp.bfloat16).astype(jnp.float32)
    return hi, a - hi


def _ln(x, g, b):
    m = jnp.mean(x, axis=-1, keepdims=True)
    v = jnp.mean((x - m) ** 2, axis=-1, keepdims=True)
    inv = 1.0 / jnp.sqrt(v + 1e-5)
    return (x - m) * inv * g + b


def _gelu(x):
    return 0.5 * x * (1.0 + jax.lax.erf(x / jnp.sqrt(jnp.float32(2.0))))


def _embed_body(x_ref, w_ref, b_ref, out_ref):
    out_ref[0] = x_ref[0] * w_ref[0] + b_ref[0]


def _embed_call(xcols, w, b):
    return pl.pallas_call(
        _embed_body,
        grid=(CT,),
        in_specs=[pl.BlockSpec((1, PADR, 1), lambda f: (f, 0, 0)),
                  pl.BlockSpec((1, 1, E), lambda f: (f, 0, 0)),
                  pl.BlockSpec((1, 1, E), lambda f: (f, 0, 0))],
        out_specs=pl.BlockSpec((1, PADR, E), lambda f: (f, 0, 0)),
        out_shape=jax.ShapeDtypeStruct((CT, PADR, E), jnp.float32),
    )(xcols, w, b)


def _fa_body(tok_ref, wq_ref, wk_ref, wv_ref, bq_ref, bk_ref, bv_ref,
             wo_ref, bo_ref, g_ref, b_ref, rc_ref, rr_ref, out_ref):
    blk = rc_ref[...] == rr_ref[...]
    li = jax.lax.broadcasted_iota(jnp.int32, (1, E), 1)
    nh = RB // HRB
    xs, qs, ks, vs = [], [], [], []
    for half in range(nh):
        x = tok_ref[:, half * HRB:(half + 1) * HRB, :].reshape(HCH, E)
        xs.append(x)
        qs.append(_dt(x, wq_ref[...]) + bq_ref[...])
        ks.append(_dt(x, wk_ref[...]) + bk_ref[...])
        vs.append(_dt(x, wv_ref[...]) + bv_ref[...])
    accs = [jnp.zeros((HCH, E), jnp.float32) for _ in range(nh)]
    for h in range(NHEAD):
        hm = ((li >= h * HD) & (li < (h + 1) * HD)).astype(jnp.float32)
        for half in range(nh):
            s = _dt(qs[half] * hm, ks[half]) * jnp.float32(1.0 / 48.0 ** 0.5)
            p = jnp.where(blk, jnp.exp(s), 0.0)
            rz = 1.0 / jnp.sum(p, axis=-1, keepdims=True)
            accs[half] = accs[half] + _dot(p, vs[half] * hm) * rz
    for half in range(nh):
        o = _dt(accs[half], wo_ref[...]) + bo_ref[...] + xs[half]
        out_ref[:, half * HRB:(half + 1) * HRB, :] = (
            _ln(o, g_ref[...], b_ref[...]).reshape(CT, HRB, E))


def _fa_call(st, wq, wk, wv, bq, bk, bv, wo, bo, g, b, rc, rr):
    def full(a):
        shp = a.shape
        return pl.BlockSpec(shp, lambda i: tuple(0 for _ in shp))
    consts = (wq, wk, wv, bq, bk, bv, wo, bo, g, b, rc, rr)
    return pl.pallas_call(
        _fa_body,
        grid=(NCH,),
        in_specs=[pl.BlockSpec((CT, RB, E), lambda i: (0, i, 0))] + [full(a) for a in consts],
        out_specs=pl.BlockSpec((CT, RB, E), lambda i: (0, i, 0)),
        out_shape=jax.ShapeDtypeStruct((CT, PADR, E), jnp.float32),
    )(st, *consts)


def _bcd_body(st_ref, qw_ref, kw_ref, ow_ref, mqw_ref, down_ref, upk_ref,
              upv_ref, mout_ref, n2g_ref, n2b_ref, n3g_ref, n3b_ref,
              m1w_ref, m1b_ref, m2w_ref, m2b_ref, out_ref, *, precise):
    x = st_ref[0][:ROWS] if PADR > ROWS else st_ref[0]
    keys = x[:SPLIT]

    c = _dt(keys, down_ref[...])
    k_all = _dt(c, upk_ref[...])
    v_all = _dt(c, upv_ref[...])
    qm = _dt(x, mqw_ref[...])
    if precise:
        qm_hi, qm_lo = _split(qm)
        ka_hi, ka_lo = _split(k_all)
        va_hi, va_lo = _split(v_all)

    qi = _dt(x, qw_ref[...])
    ki = _dt(keys, kw_ref[...])
    li = jax.lax.broadcasted_iota(jnp.int32, (1, IDXH * IDXD), 1)
    hms = [((li >= h * IDXD) & (li < (h + 1) * IDXD)).astype(jnp.float32)
           for h in range(IDXH)]
    k_cmax = jnp.max(jnp.abs(ki), axis=0, keepdims=True)
    ks = [(jnp.max(k_cmax * hms[h], keepdims=True) + 1e-6) / 127.0
          for h in range(IDXH)]
    k_rscale = sum((1.0 / ks[h]) * hms[h] for h in range(IDXH))
    k_q = jnp.clip(jnp.round(ki * k_rscale), -127.0, 127.0)

    halves = []
    for off in (0, SPLIT):
        qh = qi[off:off + SPLIT]
        q_cmax = jnp.max(jnp.abs(qh), axis=0, keepdims=True)
        qs = [(jnp.max(q_cmax * hms[h], keepdims=True) + 1e-6) / 127.0
              for h in range(IDXH)]
        q_rscale = sum((1.0 / qs[h]) * hms[h] for h in range(IDXH))
        q_q = jnp.clip(jnp.round(qh * q_rscale), -127.0, 127.0)
        red = jnp.zeros((SPLIT, SPLIT), jnp.float32)
        for h in range(IDXH):
            s = _dt(q_q * hms[h], k_q)
            red = red + (qs[h] * ks[h] * ow_ref[0:1, h:h + 1]) * jax.nn.relu(s)
        halves.append(red)
    red = jnp.concatenate(halves, axis=0)

    u = jax.lax.bitcast_convert_type(red, jnp.int32)
    key = jnp.where(u >= 0, u, jnp.int32(-2147483648) - u)
    iot = jax.lax.broadcasted_iota(jnp.int32, (ROWS, SPLIT), 1)
    cur = ((key & jnp.int32(-256)) | (jnp.int32(255) - iot)).T
    for _ in range(TOPK):
        m = jnp.max(cur, axis=0, keepdims=True)
        cur = jnp.where(cur == m, jnp.int32(-2147483648), cur)
    sel = (cur == jnp.int32(-2147483648)).T

    liE = jax.lax.broadcasted_iota(jnp.int32, (1, E), 1)
    o = jnp.zeros((ROWS, E), jnp.float32)
    for h in range(NHEAD):
        hm = ((liE >= h * HD) & (liE < (h + 1) * HD)).astype(jnp.float32)
        if precise:
            a_hi = qm_hi * hm
            a_lo = qm_lo * hm
            s = (_dt(a_hi, ka_hi) + (_dt(a_lo, ka_hi) + _dt(a_hi, ka_lo)))
        else:
            s = _dt(qm * hm, k_all)
        s = s * jnp.float32(1.0 / 48.0 ** 0.5)
        p = jnp.where(sel, jnp.exp(s), 0.0)
        rz = 1.0 / jnp.sum(p, axis=-1, keepdims=True)
        if precise:
            p_hi, p_lo = _split(p)
            vm_hi = va_hi * hm
            vm_lo = va_lo * hm
            o = o + (_dot(p_hi, vm_hi) + (_dot(p_lo, vm_hi) + _dot(p_hi, vm_lo))) * rz
        else:
            o = o + _dot(p, v_all * hm) * rz
    attn = _dt(o, mout_ref[...])
    src2 = _ln(x + attn, n2g_ref[...], n2b_ref[...])

    h1 = _gelu(_dt(src2, m1w_ref[...]) + m1b_ref[...])
    src3 = _dt(h1, m2w_ref[...]) + m2b_ref[...] + src2
    out_ref[0, :ROWS] = _ln(src3, n3g_ref[...], n3b_ref[...])
    if PADR > ROWS:
        out_ref[0, ROWS:] = jnp.zeros((PADR - ROWS, E), jnp.float32)


def _bcd_call(st, qw, kw, ow, mqw, down, upk, upv, mout,
              n2g, n2b, n3g, n3b, m1w, m1b, m2w, m2b, precise):
    def full(a):
        shp = a.shape
        return pl.BlockSpec(shp, lambda i: tuple(0 for _ in shp))
    consts = (qw, kw, ow, mqw, down, upk, upv, mout,
              n2g, n2b, n3g, n3b, m1w, m1b, m2w, m2b)
    return pl.pallas_call(
        functools.partial(_bcd_body, precise=precise),
        grid=(CT,),
        in_specs=[pl.BlockSpec((1, PADR, E), lambda f: (f, 0, 0))] + [full(a) for a in consts],
        out_specs=pl.BlockSpec((1, PADR, E), lambda f: (f, 0, 0)),
        out_shape=jax.ShapeDtypeStruct((CT, PADR, E), jnp.float32),
    )(st, *consts)


def _dec_body(x_ref, w1_ref, b1_ref, w2_ref, b2_ref, out_ref):
    h = _gelu(_dt(x_ref[...], w1_ref[...]) + b1_ref[...])
    out_ref[...] = _dt(h, w2_ref[...]) + b2_ref[...]


def _dec_call(x, w1, b1, w2, b2):
    return pl.pallas_call(
        _dec_body,
        out_shape=jax.ShapeDtypeStruct((ROWS - SPLIT, 2), jnp.float32),
    )(x, w1, b1, w2, b2)


def kernel(x_src, y_src, fa_in_w, fa_in_b, fa_out_w, fa_out_b,
           idx_qw, idx_kw, idx_ow, mla_qw, mla_down, mla_up, mla_out,
           n1_g, n1_b, n2_g, n2_b, n3_g, n3_b,
           mlp1_w, mlp1_b, mlp2_w, mlp2_b,
           fe_w, fe_b, te_w, te_b, dec1_w, dec1_b, dec2_w, dec2_b):
    f32 = jnp.float32
    y_full = jnp.concatenate([y_src[0], jnp.zeros((ROWS - SPLIT, 1), f32)], axis=0)
    xcols = jnp.pad(jnp.concatenate([x_src[0], y_full], axis=1).T,
                    ((0, 0), (0, PADR - ROWS))).reshape(CT, PADR, 1)
    w_emb = jnp.concatenate([jnp.tile(fe_w.reshape(1, E), (COLS, 1)),
                             te_w.reshape(1, E)], axis=0).reshape(CT, 1, E)
    b_emb = jnp.concatenate([jnp.tile(fe_b.reshape(1, E), (COLS, 1)),
                             te_b.reshape(1, E)], axis=0).reshape(CT, 1, E)
    st = _embed_call(xcols, w_emb, b_emb)
    rc = (jnp.arange(HCH, dtype=jnp.int32) % HRB).reshape(HCH, 1)
    rr = rc.reshape(1, HCH)
    for l in range(2):
        st = _fa_call(st,
                      fa_in_w[l, 0:E], fa_in_w[l, E:2 * E], fa_in_w[l, 2 * E:3 * E],
                      fa_in_b[l, 0:E].reshape(1, E), fa_in_b[l, E:2 * E].reshape(1, E),
                      fa_in_b[l, 2 * E:3 * E].reshape(1, E),
                      fa_out_w[l], fa_out_b[l].reshape(1, E),
                      n1_g[l].reshape(1, E), n1_b[l].reshape(1, E), rc, rr)
        st = _bcd_call(st, idx_qw[l], idx_kw[l], idx_ow[l].reshape(1, IDXH),
                       mla_qw[l], mla_down[l], mla_up[l][:E], mla_up[l][E:],
                       mla_out[l],
                       n2_g[l].reshape(1, E), n2_b[l].reshape(1, E),
                       n3_g[l].reshape(1, E), n3_b[l].reshape(1, E),
                       mlp1_w[l], mlp1_b[l].reshape(1, MLPD),
                       mlp2_w[l], mlp2_b[l].reshape(1, E), precise=(l == 0))
    tgt = st[CT - 1, SPLIT:ROWS]
    out = _dec_call(tgt, dec1_w, dec1_b.reshape(1, MLPD),
                    dec2_w, dec2_b.reshape(1, 2))
    return out.reshape(1, ROWS - SPLIT, 2)
